# baseline (speedup 1.0000x reference)
import functools

import jax
import jax.numpy as jnp
from jax import lax
from jax.experimental import pallas as pl
from jax.experimental.pallas import tpu as pltpu
from jax.experimental.pallas import tpu_sc as plsc

VOCAB = 1000000
DIM = 64
BATCH = 16384
HIST = 50

_B = BATCH * HIST

_info = plsc.get_sparse_core_info()
_NC, _NS = _info.num_cores, _info.num_subcores
_NW = _NC * _NS

_B_PER_W = _B // _NW
_CHUNK = 800
_N_CHUNK = _B_PER_W // _CHUNK
_N_OUTER = _N_CHUNK // 2

_mesh = plsc.VectorSubcoreMesh(core_axis_name="c", subcore_axis_name="s")


@functools.partial(
    pl.kernel,
    mesh=_mesh,
    out_type=jax.ShapeDtypeStruct((_B, DIM), jnp.float32),
    scratch_types=[
        pltpu.VMEM((_B_PER_W,), jnp.int32),
        pltpu.VMEM((_CHUNK, DIM), jnp.float32),
        pltpu.VMEM((_CHUNK, DIM), jnp.float32),
        pltpu.SemaphoreType.DMA,
        pltpu.SemaphoreType.DMA,
    ],
    compiler_params=pltpu.CompilerParams(use_tc_tiling_on_sc=False),
)
def _gather_kernel(table_hbm, idx_hbm, out_hbm, idx_v, rows0, rows1, sem0, sem1):
    wid = lax.axis_index("s") * _NC + lax.axis_index("c")
    base = wid * _B_PER_W

    pltpu.sync_copy(idx_hbm.at[pl.ds(base, _B_PER_W)], idx_v)

    def start_gather(s, rows, sem):
        pltpu.async_copy(table_hbm.at[idx_v.at[pl.ds(s * _CHUNK, _CHUNK)]],
                         rows, sem)

    def finish(s, rows, sem):
        pltpu.make_async_copy(
            table_hbm.at[idx_v.at[pl.ds(s * _CHUNK, _CHUNK)]], rows, sem
        ).wait()
        pltpu.sync_copy(rows, out_hbm.at[pl.ds(base + s * _CHUNK, _CHUNK)])

    start_gather(0, rows0, sem0)

    def outer(o, _):
        s0 = 2 * o
        start_gather(s0 + 1, rows1, sem1)
        finish(s0, rows0, sem0)
        start_gather(s0 + 2, rows0, sem0)
        finish(s0 + 1, rows1, sem1)
        return _

    lax.fori_loop(0, _N_OUTER - 1, outer, None)

    s0 = _N_CHUNK - 2
    start_gather(s0 + 1, rows1, sem1)
    finish(s0, rows0, sem0)
    finish(s0 + 1, rows1, sem1)


_K2_BW = 2048
_K2_NB = BATCH // _K2_BW


def _eye64():
    return (lax.broadcasted_iota(jnp.int32, (DIM, DIM), 0)
            == lax.broadcasted_iota(jnp.int32, (DIM, DIM), 1)).astype(jnp.float32)


def _k2_body(g_ref, o_ref):
    eye = _eye64()
    dn = (((0,), (1,)), ((), ()))
    o_ref[0, :, 0:_K2_BW // 2] = lax.dot_general(
        eye, g_ref[:, 0:DIM], dn, preferred_element_type=jnp.float32)
    o_ref[0, :, _K2_BW // 2:_K2_BW] = lax.dot_general(
        eye, g_ref[:, DIM:128], dn, preferred_element_type=jnp.float32)


_k2_call = pl.pallas_call(
    _k2_body,
    grid=(HIST, _K2_NB),
    in_specs=[pl.BlockSpec((_K2_BW // 2, 128),
                           lambda h, jb: (h * _K2_NB + jb, 0))],
    out_specs=pl.BlockSpec((1, DIM, _K2_BW), lambda h, jb: (h, 0, jb)),
    out_shape=jax.ShapeDtypeStruct((HIST, DIM, BATCH), jnp.float32),
)


def kernel(x, embedding):
    xperm = x.T.reshape(HIST, _K2_NB, 2, _K2_BW // 2)
    xperm = xperm.transpose(0, 1, 3, 2).reshape(_B)
    g = _gather_kernel(embedding, xperm)
    g2 = g.reshape(_B // 2, 128)
    outT = _k2_call(g2)
    return outT.transpose(2, 0, 1)
